# Initial kernel scaffold; baseline (speedup 1.0000x reference)
import jax
import jax.numpy as jnp
from jax.experimental import pallas as pl


def kernel(features, permuted_feature, edge_index, W1, b1, W2, b2, Wp, bp):
    raise NotImplementedError("write your pallas kernel here")


# speedup vs baseline: 3.3308x; 3.3308x over previous
import functools

import jax
import jax.numpy as jnp
from jax import lax
from jax.experimental import pallas as pl
from jax.experimental.pallas import tpu as pltpu
from jax.experimental.pallas import tpu_sc as plsc

N = 10000
NPAD = 10240
D = 128
K = 128
NC = 2
NS = 16


def _z16():
    return jnp.zeros((16,), jnp.float32)


def _degrees_sc(ep, comb):
    et = ep // NS
    nch = et // K
    mesh = plsc.VectorSubcoreMesh(core_axis_name="c", subcore_axis_name="s")

    @functools.partial(
        pl.kernel,
        out_type=jax.ShapeDtypeStruct((NC * NPAD, D), jnp.float32),
        mesh=mesh,
        scratch_types=[
            pltpu.VMEM((K,), jnp.int32),
            pltpu.VMEM((K, D), jnp.float32),
            pltpu.VMEM((K, D), jnp.float32),
            pltpu.VMEM_SHARED((NPAD, D), jnp.float32),
        ],
    )
    def deg_kernel(comb_hbm, out_hbm, idxb, valb, zb, acc):
        c = lax.axis_index("c")
        s = lax.axis_index("s")
        lane0 = jnp.maximum(1.0 - lax.iota(jnp.int32, 16).astype(jnp.float32),
                            0.0)

        def fill(i, _):
            valb[i, pl.ds(0, 16)] = lane0
            for t in range(1, D // 16):
                valb[i, pl.ds(t * 16, 16)] = _z16()
            for t in range(D // 16):
                zb[i, pl.ds(t * 16, 16)] = _z16()
            return 0

        lax.fori_loop(0, K, fill, 0)
        zrows = NPAD // NS

        def zloop(t, _):
            pltpu.sync_copy(zb, acc.at[pl.ds(s * zrows + t * K, K)])
            return 0

        lax.fori_loop(0, zrows // K, zloop, 0)
        plsc.subcore_barrier()

        def chunk(ch, _):
            base = c * ep + s * et + ch * K
            pltpu.sync_copy(comb_hbm.at[pl.ds(base, K)], idxb)
            pltpu.sync_copy(valb, acc.at[idxb], add=True)
            return 0

        lax.fori_loop(0, nch, chunk, 0)
        plsc.subcore_barrier()
        pltpu.sync_copy(acc.at[pl.ds(s * zrows, zrows)],
                        out_hbm.at[pl.ds(c * NPAD + s * zrows, zrows)])

    return deg_kernel(comb)


def _scale_tc(feats2, dego, degi):
    B = 256
    nb = NPAD // B

    def body(f_ref, do_ref, di_ref, xs_ref, aux_ref, dis_ref):
        dos = lax.rsqrt(jnp.maximum(do_ref[:, 0:1], 1.0))
        dis = lax.rsqrt(jnp.maximum(di_ref[:, 0:1], 1.0))
        xs_ref[...] = f_ref[...] * dos
        lane = lax.broadcasted_iota(jnp.int32, (B, D), 1)
        aux_ref[...] = jnp.where(lane == 0, dos,
                                 jnp.where(lane == 1, dis, 0.0))
        dis_ref[...] = jnp.broadcast_to(dis, (B, D))

    return pl.pallas_call(
        body,
        grid=(2 * nb,),
        in_specs=[
            pl.BlockSpec((B, D), lambda i: (i, 0)),
            pl.BlockSpec((B, D), lambda i: (lax.rem(i, nb), 0)),
            pl.BlockSpec((B, D), lambda i: (NPAD // B + lax.rem(i, nb), 0)),
        ],
        out_specs=[
            pl.BlockSpec((B, D), lambda i: (i, 0)),
            pl.BlockSpec((B, D), lambda i: (lax.rem(i, nb), 0)),
            pl.BlockSpec((B, D), lambda i: (lax.rem(i, nb), 0)),
        ],
        out_shape=[
            jax.ShapeDtypeStruct((2 * NPAD, D), jnp.float32),
            jax.ShapeDtypeStruct((NPAD, D), jnp.float32),
            jax.ShapeDtypeStruct((NPAD, D), jnp.float32),
        ],
    )(feats2, dego, degi)


def _w_sc(ep, srcp, dstp, distab):
    et = ep // (NC * NS)
    nch = et // K
    mesh = plsc.VectorSubcoreMesh(core_axis_name="c", subcore_axis_name="s")

    @functools.partial(
        pl.kernel,
        out_type=jax.ShapeDtypeStruct((NC * NPAD, D), jnp.float32),
        mesh=mesh,
        scratch_types=[
            pltpu.VMEM((K,), jnp.int32),
            pltpu.VMEM((K,), jnp.int32),
            pltpu.VMEM((K, D), jnp.float32),
            pltpu.SemaphoreType.DMA,
            pltpu.VMEM_SHARED((NPAD, D), jnp.float32),
        ],
    )
    def w_kernel(src_hbm, dst_hbm, dis_hbm, out_hbm, sidx, gidx, rowsb, sem,
                 acc):
        c = lax.axis_index("c")
        s = lax.axis_index("s")

        def fill(i, _):
            for t in range(D // 16):
                rowsb[i, pl.ds(t * 16, 16)] = _z16()
            return 0

        lax.fori_loop(0, K, fill, 0)
        zrows = NPAD // NS

        def zloop(t, _):
            pltpu.sync_copy(rowsb, acc.at[pl.ds(s * zrows + t * K, K)])
            return 0

        lax.fori_loop(0, zrows // K, zloop, 0)
        plsc.subcore_barrier()

        def chunk(ch, _):
            base = c * (ep // NC) + s * et + ch * K
            pltpu.sync_copy(dst_hbm.at[pl.ds(base, K)], gidx)
            pltpu.sync_copy(src_hbm.at[pl.ds(base, K)], sidx)
            pltpu.async_copy(dis_hbm.at[gidx], rowsb, sem).wait()
            pltpu.sync_copy(rowsb, acc.at[sidx], add=True)
            return 0

        lax.fori_loop(0, nch, chunk, 0)
        plsc.subcore_barrier()
        pltpu.sync_copy(acc.at[pl.ds(s * zrows, zrows)],
                        out_hbm.at[pl.ds(c * NPAD + s * zrows, zrows)])

    return w_kernel(srcp, dstp, distab)


def _aggregate_sc(ep, xs, srcp, dstp):
    et = ep // NS
    nch = et // K
    mesh = plsc.VectorSubcoreMesh(core_axis_name="c", subcore_axis_name="s")

    @functools.partial(
        pl.kernel,
        out_type=jax.ShapeDtypeStruct((NC * NPAD, D), jnp.float32),
        mesh=mesh,
        scratch_types=[
            pltpu.VMEM((K,), jnp.int32),
            pltpu.VMEM((K,), jnp.int32),
            pltpu.VMEM((K, D), jnp.float32),
            pltpu.SemaphoreType.DMA,
            pltpu.VMEM_SHARED((NPAD, D), jnp.float32),
        ],
    )
    def agg_kernel(xs_hbm, src_hbm, dst_hbm, agg_hbm, sidx, didx, rowsb, sem,
                   acc):
        c = lax.axis_index("c")
        s = lax.axis_index("s")
        off = c * NPAD

        def fill(i, _):
            for t in range(D // 16):
                rowsb[i, pl.ds(t * 16, 16)] = _z16()
            return 0

        lax.fori_loop(0, K, fill, 0)
        zrows = NPAD // NS

        def zloop(t, _):
            pltpu.sync_copy(rowsb, acc.at[pl.ds(s * zrows + t * K, K)])
            return 0

        lax.fori_loop(0, zrows // K, zloop, 0)
        plsc.subcore_barrier()

        def chunk(ch, _):
            base = s * et + ch * K
            pltpu.sync_copy(src_hbm.at[pl.ds(base, K)], sidx)
            pltpu.sync_copy(dst_hbm.at[pl.ds(base, K)], didx)
            for g in range(K // 16):
                sl = pl.ds(g * 16, 16)
                sidx[sl] = sidx[sl] + off
            pltpu.async_copy(xs_hbm.at[sidx], rowsb, sem).wait()
            pltpu.sync_copy(rowsb, acc.at[didx], add=True)
            return 0

        lax.fori_loop(0, nch, chunk, 0)
        plsc.subcore_barrier()
        pltpu.sync_copy(acc.at[pl.ds(s * zrows, zrows)],
                        agg_hbm.at[pl.ds(c * NPAD + s * zrows, zrows)])

    return agg_kernel(xs, srcp, dstp)


def _reduce_tc(agg, aux, wpart, W1, b1r, W2, b2r, Wp, bpr):
    B = 512
    nb = NPAD // B
    nsteps = 2 * nb

    def body(a_ref, aux_ref, w1_ref, w2_ref, W1_ref, b1_ref, W2_ref, b2_ref,
             Wp_ref, bp_ref, out_ref, vacc):
        i = pl.program_id(0)
        inp = i // nb
        blk = lax.rem(i, nb)

        @pl.when(i == 0)
        def _():
            vacc[...] = jnp.zeros((8, D), jnp.float32)

        z = jnp.dot(a_ref[...], W1_ref[...], preferred_element_type=jnp.float32)
        dos = aux_ref[:, 0:1]
        dis = aux_ref[:, 1:2]
        rh = jnp.maximum(z * dis + b1_ref[...], 0.0)
        wcol = w1_ref[:, 0:1] + w2_ref[:, 0:1]
        rowid = lax.broadcasted_iota(jnp.int32, (B, 1), 0) + blk * B
        cc = jnp.where(rowid < N, wcol * dos, 0.0)
        part = jnp.sum(rh * cc, axis=0, keepdims=True)
        sl = pl.ds(inp, 1)
        vacc[sl, :] = vacc[sl, :] + part

        @pl.when(i == nsteps - 1)
        def _():
            u = jnp.sum(Wp_ref[...], axis=1, keepdims=True)
            q = jnp.dot(W2_ref[...], u, preferred_element_type=jnp.float32)
            base = (jnp.dot(b2_ref[...], u, preferred_element_type=jnp.float32)
                    * float(N)
                    + jnp.sum(bp_ref[...], keepdims=True).reshape(1, 1)
                    * float(N))
            s1 = jnp.dot(vacc[0:1, :], q, preferred_element_type=jnp.float32) + base
            s2 = jnp.dot(vacc[1:2, :], q, preferred_element_type=jnp.float32) + base
            l1 = jnp.maximum(s1, 0.0) - s1 + jnp.log1p(jnp.exp(-jnp.abs(s1)))
            l2 = jnp.maximum(s2, 0.0) + jnp.log1p(jnp.exp(-jnp.abs(s2)))
            out_ref[...] = 0.5 * (l1 + l2)

    return pl.pallas_call(
        body,
        grid=(nsteps,),
        in_specs=[
            pl.BlockSpec((B, D), lambda i: (i, 0)),
            pl.BlockSpec((B, D), lambda i: (lax.rem(i, nb), 0)),
            pl.BlockSpec((B, D), lambda i: (lax.rem(i, nb), 0)),
            pl.BlockSpec((B, D), lambda i: (NPAD // B + lax.rem(i, nb), 0)),
            pl.BlockSpec((D, D), lambda i: (0, 0)),
            pl.BlockSpec((1, D), lambda i: (0, 0)),
            pl.BlockSpec((D, D), lambda i: (0, 0)),
            pl.BlockSpec((1, D), lambda i: (0, 0)),
            pl.BlockSpec((D, D), lambda i: (0, 0)),
            pl.BlockSpec((1, D), lambda i: (0, 0)),
        ],
        out_specs=pl.BlockSpec((1, 1), lambda i: (0, 0)),
        out_shape=jax.ShapeDtypeStruct((1, 1), jnp.float32),
        scratch_shapes=[pltpu.VMEM((8, D), jnp.float32)],
    )(agg, aux, wpart, wpart, W1, b1r, W2, b2r, Wp, bpr)


def kernel(features, permuted_feature, edge_index, W1, b1, W2, b2, Wp, bp):
    E = edge_index.shape[1]
    EP = -(-E // (NC * NS * K)) * (NC * NS * K)

    pad = jnp.full((EP - E,), N, jnp.int32)
    srcp = jnp.concatenate([edge_index[0], pad])
    dstp = jnp.concatenate([edge_index[1], pad])
    comb = jnp.concatenate([srcp, dstp])

    degs = _degrees_sc(EP, comb)

    fp = jnp.zeros((2 * NPAD, D), jnp.float32)
    fp = lax.dynamic_update_slice(fp, features, (0, 0))
    fp = lax.dynamic_update_slice(fp, permuted_feature, (NPAD, 0))

    xs, aux, distab = _scale_tc(fp, degs, degs)
    wpart = _w_sc(EP, srcp, dstp, distab)
    agg = _aggregate_sc(EP, xs, srcp, dstp)

    loss = _reduce_tc(agg, aux, wpart, W1, b1.reshape(1, D), W2,
                      b2.reshape(1, D), Wp, bp.reshape(1, D))
    return loss[0, 0]


# speedup vs baseline: 3.7209x; 1.1171x over previous
import functools

import jax
import jax.numpy as jnp
from jax import lax
from jax.experimental import pallas as pl
from jax.experimental.pallas import tpu as pltpu
from jax.experimental.pallas import tpu_sc as plsc

N = 10000
NPAD = 10240
D = 128
K = 128
NC = 2
NS = 16


def _z16():
    return jnp.zeros((16,), jnp.float32)


def _degrees_sc(ep, comb):
    et = ep // NS
    nch = et // K
    mesh = plsc.VectorSubcoreMesh(core_axis_name="c", subcore_axis_name="s")

    @functools.partial(
        pl.kernel,
        out_type=jax.ShapeDtypeStruct((NC * NPAD, D), jnp.float32),
        mesh=mesh,
        scratch_types=[
            pltpu.VMEM((K,), jnp.int32),
            pltpu.VMEM((K, D), jnp.float32),
            pltpu.VMEM((K, D), jnp.float32),
            pltpu.VMEM_SHARED((NPAD, D), jnp.float32),
        ],
    )
    def deg_kernel(comb_hbm, out_hbm, idxb, valb, zb, acc):
        c = lax.axis_index("c")
        s = lax.axis_index("s")
        lane0 = jnp.maximum(1.0 - lax.iota(jnp.int32, 16).astype(jnp.float32),
                            0.0)

        def fill(i, _):
            valb[i, pl.ds(0, 16)] = lane0
            for t in range(1, D // 16):
                valb[i, pl.ds(t * 16, 16)] = _z16()
            for t in range(D // 16):
                zb[i, pl.ds(t * 16, 16)] = _z16()
            return 0

        lax.fori_loop(0, K, fill, 0)
        zrows = NPAD // NS

        def zloop(t, _):
            pltpu.sync_copy(zb, acc.at[pl.ds(s * zrows + t * K, K)])
            return 0

        lax.fori_loop(0, zrows // K, zloop, 0)
        plsc.subcore_barrier()

        def chunk(ch, _):
            base = c * ep + s * et + ch * K
            pltpu.sync_copy(comb_hbm.at[pl.ds(base, K)], idxb)
            pltpu.sync_copy(valb, acc.at[idxb], add=True)
            return 0

        lax.fori_loop(0, nch, chunk, 0)
        plsc.subcore_barrier()
        pltpu.sync_copy(acc.at[pl.ds(s * zrows, zrows)],
                        out_hbm.at[pl.ds(c * NPAD + s * zrows, zrows)])

    return deg_kernel(comb)


def _scale_tc(feats2, dego, degi):
    B = 256
    nb = NPAD // B

    def body(f_ref, do_ref, di_ref, xs_ref, aux_ref, dis_ref):
        dos = lax.rsqrt(jnp.maximum(do_ref[:, 0:1], 1.0))
        dis = lax.rsqrt(jnp.maximum(di_ref[:, 0:1], 1.0))
        xs_ref[...] = f_ref[...] * dos
        lane = lax.broadcasted_iota(jnp.int32, (B, D), 1)
        aux_ref[...] = jnp.where(lane == 0, dos,
                                 jnp.where(lane == 1, dis, 0.0))
        dis_ref[...] = jnp.broadcast_to(dis, (B, D))

    return pl.pallas_call(
        body,
        grid=(2 * nb,),
        in_specs=[
            pl.BlockSpec((B, D), lambda i: (i, 0)),
            pl.BlockSpec((B, D), lambda i: (lax.rem(i, nb), 0)),
            pl.BlockSpec((B, D), lambda i: (NPAD // B + lax.rem(i, nb), 0)),
        ],
        out_specs=[
            pl.BlockSpec((B, D), lambda i: (i, 0)),
            pl.BlockSpec((B, D), lambda i: (lax.rem(i, nb), 0)),
            pl.BlockSpec((B, D), lambda i: (lax.rem(i, nb), 0)),
        ],
        out_shape=[
            jax.ShapeDtypeStruct((2 * NPAD, D), jnp.float32),
            jax.ShapeDtypeStruct((NPAD, D), jnp.float32),
            jax.ShapeDtypeStruct((NPAD, D), jnp.float32),
        ],
    )(feats2, dego, degi)


def _w_sc(ep, srcp, dstp, distab):
    et = ep // (NC * NS)
    nch = et // K
    mesh = plsc.VectorSubcoreMesh(core_axis_name="c", subcore_axis_name="s")

    @functools.partial(
        pl.kernel,
        out_type=jax.ShapeDtypeStruct((NC * NPAD, D), jnp.float32),
        mesh=mesh,
        scratch_types=[
            pltpu.VMEM((K,), jnp.int32),
            pltpu.VMEM((K,), jnp.int32),
            pltpu.VMEM((K, D), jnp.float32),
            pltpu.SemaphoreType.DMA,
            pltpu.VMEM_SHARED((NPAD, D), jnp.float32),
        ],
    )
    def w_kernel(src_hbm, dst_hbm, dis_hbm, out_hbm, sidx, gidx, rowsb, sem,
                 acc):
        c = lax.axis_index("c")
        s = lax.axis_index("s")

        def fill(i, _):
            for t in range(D // 16):
                rowsb[i, pl.ds(t * 16, 16)] = _z16()
            return 0

        lax.fori_loop(0, K, fill, 0)
        zrows = NPAD // NS

        def zloop(t, _):
            pltpu.sync_copy(rowsb, acc.at[pl.ds(s * zrows + t * K, K)])
            return 0

        lax.fori_loop(0, zrows // K, zloop, 0)
        plsc.subcore_barrier()

        def chunk(ch, _):
            base = c * (ep // NC) + s * et + ch * K
            pltpu.sync_copy(dst_hbm.at[pl.ds(base, K)], gidx)
            pltpu.sync_copy(src_hbm.at[pl.ds(base, K)], sidx)
            pltpu.async_copy(dis_hbm.at[gidx], rowsb, sem).wait()
            pltpu.sync_copy(rowsb, acc.at[sidx], add=True)
            return 0

        lax.fori_loop(0, nch, chunk, 0)
        plsc.subcore_barrier()
        pltpu.sync_copy(acc.at[pl.ds(s * zrows, zrows)],
                        out_hbm.at[pl.ds(c * NPAD + s * zrows, zrows)])

    return w_kernel(srcp, dstp, distab)


def _aggregate_sc(ep, xs, srcp, dstp):
    et = ep // NS
    nch = et // K
    mesh = plsc.VectorSubcoreMesh(core_axis_name="c", subcore_axis_name="s")

    @functools.partial(
        pl.kernel,
        out_type=jax.ShapeDtypeStruct((NC * NPAD, D), jnp.float32),
        mesh=mesh,
        scratch_types=[
            pltpu.VMEM((K,), jnp.int32),
            pltpu.VMEM((K,), jnp.int32),
            pltpu.VMEM((K,), jnp.int32),
            pltpu.VMEM((K,), jnp.int32),
            pltpu.VMEM((K, D), jnp.float32),
            pltpu.VMEM((K, D), jnp.float32),
            pltpu.SemaphoreType.DMA,
            pltpu.SemaphoreType.DMA,
            pltpu.VMEM_SHARED((NPAD, D), jnp.float32),
        ],
    )
    def agg_kernel(xs_hbm, src_hbm, dst_hbm, agg_hbm, sidx0, didx0, sidx1,
                   didx1, rows0, rows1, sem0, sem1, acc):
        c = lax.axis_index("c")
        s = lax.axis_index("s")
        off = c * NPAD
        sb = (sidx0, sidx1)
        db = (didx0, didx1)
        rb = (rows0, rows1)
        sm = (sem0, sem1)

        def fill(i, _):
            for t in range(D // 16):
                rows0[i, pl.ds(t * 16, 16)] = _z16()
            return 0

        lax.fori_loop(0, K, fill, 0)
        zrows = NPAD // NS

        def zloop(t, _):
            pltpu.sync_copy(rows0, acc.at[pl.ds(s * zrows + t * K, K)])
            return 0

        lax.fori_loop(0, zrows // K, zloop, 0)
        plsc.subcore_barrier()

        def chunk(m, _):
            cps = [None, None]
            for b in range(2):
                base = s * et + (m * 2 + b) * K
                pltpu.sync_copy(src_hbm.at[pl.ds(base, K)], sb[b])
                pltpu.sync_copy(dst_hbm.at[pl.ds(base, K)], db[b])
                for g in range(K // 16):
                    sl = pl.ds(g * 16, 16)
                    sb[b][sl] = sb[b][sl] + off
                cps[b] = pltpu.async_copy(xs_hbm.at[sb[b]], rb[b], sm[b])
            for b in range(2):
                cps[b].wait()
                pltpu.sync_copy(rb[b], acc.at[db[b]], add=True)
            return 0

        lax.fori_loop(0, nch // 2, chunk, 0)
        plsc.subcore_barrier()
        pltpu.sync_copy(acc.at[pl.ds(s * zrows, zrows)],
                        agg_hbm.at[pl.ds(c * NPAD + s * zrows, zrows)])

    return agg_kernel(xs, srcp, dstp)


def _reduce_tc(agg, aux, wpart, W1, b1r, W2, b2r, Wp, bpr):
    B = 512
    nb = NPAD // B
    nsteps = 2 * nb

    def body(a_ref, aux_ref, w1_ref, w2_ref, W1_ref, b1_ref, W2_ref, b2_ref,
             Wp_ref, bp_ref, out_ref, vacc):
        i = pl.program_id(0)
        inp = i // nb
        blk = lax.rem(i, nb)

        @pl.when(i == 0)
        def _():
            vacc[...] = jnp.zeros((8, D), jnp.float32)

        z = jnp.dot(a_ref[...], W1_ref[...], preferred_element_type=jnp.float32)
        dos = aux_ref[:, 0:1]
        dis = aux_ref[:, 1:2]
        rh = jnp.maximum(z * dis + b1_ref[...], 0.0)
        wcol = w1_ref[:, 0:1] + w2_ref[:, 0:1]
        rowid = lax.broadcasted_iota(jnp.int32, (B, 1), 0) + blk * B
        cc = jnp.where(rowid < N, wcol * dos, 0.0)
        part = jnp.sum(rh * cc, axis=0, keepdims=True)
        sl = pl.ds(inp, 1)
        vacc[sl, :] = vacc[sl, :] + part

        @pl.when(i == nsteps - 1)
        def _():
            u = jnp.sum(Wp_ref[...], axis=1, keepdims=True)
            q = jnp.dot(W2_ref[...], u, preferred_element_type=jnp.float32)
            base = (jnp.dot(b2_ref[...], u, preferred_element_type=jnp.float32)
                    * float(N)
                    + jnp.sum(bp_ref[...], keepdims=True).reshape(1, 1)
                    * float(N))
            s1 = jnp.dot(vacc[0:1, :], q, preferred_element_type=jnp.float32) + base
            s2 = jnp.dot(vacc[1:2, :], q, preferred_element_type=jnp.float32) + base
            l1 = jnp.maximum(s1, 0.0) - s1 + jnp.log1p(jnp.exp(-jnp.abs(s1)))
            l2 = jnp.maximum(s2, 0.0) + jnp.log1p(jnp.exp(-jnp.abs(s2)))
            out_ref[...] = 0.5 * (l1 + l2)

    return pl.pallas_call(
        body,
        grid=(nsteps,),
        in_specs=[
            pl.BlockSpec((B, D), lambda i: (i, 0)),
            pl.BlockSpec((B, D), lambda i: (lax.rem(i, nb), 0)),
            pl.BlockSpec((B, D), lambda i: (lax.rem(i, nb), 0)),
            pl.BlockSpec((B, D), lambda i: (NPAD // B + lax.rem(i, nb), 0)),
            pl.BlockSpec((D, D), lambda i: (0, 0)),
            pl.BlockSpec((1, D), lambda i: (0, 0)),
            pl.BlockSpec((D, D), lambda i: (0, 0)),
            pl.BlockSpec((1, D), lambda i: (0, 0)),
            pl.BlockSpec((D, D), lambda i: (0, 0)),
            pl.BlockSpec((1, D), lambda i: (0, 0)),
        ],
        out_specs=pl.BlockSpec((1, 1), lambda i: (0, 0)),
        out_shape=jax.ShapeDtypeStruct((1, 1), jnp.float32),
        scratch_shapes=[pltpu.VMEM((8, D), jnp.float32)],
    )(agg, aux, wpart, wpart, W1, b1r, W2, b2r, Wp, bpr)


def kernel(features, permuted_feature, edge_index, W1, b1, W2, b2, Wp, bp):
    E = edge_index.shape[1]
    EP = -(-E // (NC * NS * K)) * (NC * NS * K)

    pad = jnp.full((EP - E,), N, jnp.int32)
    srcp = jnp.concatenate([edge_index[0], pad])
    dstp = jnp.concatenate([edge_index[1], pad])
    comb = jnp.concatenate([srcp, dstp])

    degs = _degrees_sc(EP, comb)

    fp = jnp.zeros((2 * NPAD, D), jnp.float32)
    fp = lax.dynamic_update_slice(fp, features, (0, 0))
    fp = lax.dynamic_update_slice(fp, permuted_feature, (NPAD, 0))

    xs, aux, distab = _scale_tc(fp, degs, degs)
    wpart = _w_sc(EP, srcp, dstp, distab)
    agg = _aggregate_sc(EP, xs, srcp, dstp)

    loss = _reduce_tc(agg, aux, wpart, W1, b1.reshape(1, D), W2,
                      b2.reshape(1, D), Wp, bp.reshape(1, D))
    return loss[0, 0]
